# baseline (speedup 1.0000x reference)
import jax
import jax.numpy as jnp
from jax import lax
from jax.experimental import pallas as pl
from jax.experimental.pallas import tpu as pltpu

B_USERS = 256
B_ITEMS = 256
HIDDEN_DIM = 128
N_ROWS = 1024
BLK = 256
K_STEPS = N_ROWS // BLK


def _body(uid_ref, iid_ref, utab_ref, itab_ref, o_ref, u_acc, v_acc):
  k = pl.program_id(0)
  uid = uid_ref[0]
  iid = iid_ref[0]
  rows = k * BLK + lax.broadcasted_iota(jnp.int32, (B_USERS, BLK), 1)
  pu = (uid[:, None] == rows).astype(jnp.float32)
  pv = (iid[:, None] == rows).astype(jnp.float32)
  du = jnp.dot(pu, utab_ref[...], preferred_element_type=jnp.float32)
  dv = jnp.dot(pv, itab_ref[...], preferred_element_type=jnp.float32)

  @pl.when(k == 0)
  def _():
    u_acc[...] = du
    v_acc[...] = dv

  @pl.when(jnp.logical_and(k > 0, k < K_STEPS - 1))
  def _():
    u_acc[...] += du
    v_acc[...] += dv

  @pl.when(k == K_STEPS - 1)
  def _():
    o_ref[...] = lax.dot_general(
        u_acc[...] + du, v_acc[...] + dv,
        dimension_numbers=(((1,), (1,)), ((), ())),
        preferred_element_type=jnp.float32)


_call = pl.pallas_call(
    _body,
    grid=(K_STEPS,),
    in_specs=[
        pl.BlockSpec((1, B_USERS), lambda k: (0, 0)),
        pl.BlockSpec((1, B_ITEMS), lambda k: (0, 0)),
        pl.BlockSpec((BLK, HIDDEN_DIM), lambda k: (k, 0)),
        pl.BlockSpec((BLK, HIDDEN_DIM), lambda k: (k, 0)),
    ],
    out_specs=pl.BlockSpec((B_USERS, B_ITEMS), lambda k: (0, 0)),
    out_shape=jax.ShapeDtypeStruct((B_USERS, B_ITEMS), jnp.float32),
    scratch_shapes=[
        pltpu.VMEM((B_USERS, HIDDEN_DIM), jnp.float32),
        pltpu.VMEM((B_ITEMS, HIDDEN_DIM), jnp.float32),
    ],
)


@jax.jit
def kernel(user_ids, item_ids, user_table, item_table):
  return _call(user_ids.reshape(1, B_USERS), item_ids.reshape(1, B_ITEMS),
               user_table, item_table)


# speedup vs baseline: 1.1166x; 1.1166x over previous
import jax
import jax.numpy as jnp
from jax import lax
from jax.experimental import pallas as pl
from jax.experimental.pallas import tpu as pltpu

B_USERS = 256
B_ITEMS = 256
HIDDEN_DIM = 128
N_ROWS = 1024


def _body(uid_ref, iid_ref, utab_hbm, itab_ref, o_ref, utab_v, sem_u):
  cu = pltpu.make_async_copy(utab_hbm, utab_v, sem_u)
  cu.start()
  uid = uid_ref[0]
  iid = iid_ref[0]
  rows = lax.broadcasted_iota(jnp.int32, (B_USERS, N_ROWS), 1)
  pv = (iid[:, None] == rows).astype(jnp.float32)
  v = jnp.dot(pv, itab_ref[...], preferred_element_type=jnp.float32)
  pu = (uid[:, None] == rows).astype(jnp.float32)
  cu.wait()
  u = jnp.dot(pu, utab_v[...], preferred_element_type=jnp.float32)
  o_ref[...] = lax.dot_general(
      u, v, dimension_numbers=(((1,), (1,)), ((), ())),
      preferred_element_type=jnp.float32)


_call = pl.pallas_call(
    _body,
    in_specs=[
        pl.BlockSpec((1, B_USERS), lambda: (0, 0)),
        pl.BlockSpec((1, B_ITEMS), lambda: (0, 0)),
        pl.BlockSpec(memory_space=pl.ANY),
        pl.BlockSpec((N_ROWS, HIDDEN_DIM), lambda: (0, 0)),
    ],
    out_specs=pl.BlockSpec((B_USERS, B_ITEMS), lambda: (0, 0)),
    out_shape=jax.ShapeDtypeStruct((B_USERS, B_ITEMS), jnp.float32),
    scratch_shapes=[
        pltpu.VMEM((N_ROWS, HIDDEN_DIM), jnp.float32),
        pltpu.SemaphoreType.DMA,
    ],
)


@jax.jit
def kernel(user_ids, item_ids, user_table, item_table):
  return _call(user_ids.reshape(1, B_USERS), item_ids.reshape(1, B_ITEMS),
               user_table, item_table)


# speedup vs baseline: 1.4570x; 1.3049x over previous
import jax
import jax.numpy as jnp
from jax import lax
from jax.experimental import pallas as pl

B_USERS = 256
B_ITEMS = 256
HIDDEN_DIM = 128
N_ROWS = 1024


def _body(uid_ref, iid_ref, utab_ref, itab_ref, o_ref):
  uid = uid_ref[0]
  iid = iid_ref[0]
  rows = lax.broadcasted_iota(jnp.int32, (B_USERS, N_ROWS), 1)
  pu = (uid[:, None] == rows).astype(jnp.float32)
  pv = (iid[:, None] == rows).astype(jnp.float32)
  u = jnp.dot(pu, utab_ref[...], preferred_element_type=jnp.float32)
  v = jnp.dot(pv, itab_ref[...], preferred_element_type=jnp.float32)
  o_ref[...] = lax.dot_general(
      u, v, dimension_numbers=(((1,), (1,)), ((), ())),
      preferred_element_type=jnp.float32)


_call = pl.pallas_call(
    _body,
    out_shape=jax.ShapeDtypeStruct((B_USERS, B_ITEMS), jnp.float32),
)


@jax.jit
def kernel(user_ids, item_ids, user_table, item_table):
  return _call(user_ids.reshape(1, B_USERS), item_ids.reshape(1, B_ITEMS),
               user_table, item_table)
